# baseline (device time: 11291 ns/iter reference)
import jax
import jax.numpy as jnp
from jax import lax
from jax.experimental import pallas as pl
from jax.experimental.pallas import tpu as pltpu

N_DEV = 4
E_LOCAL = 2


def kernel(x, router_W, route_idx, expert_W):
    n_tok, d = x.shape
    h = expert_W.shape[-1]
    n_exp = router_W.shape[-1]
    q = n_tok // N_DEV

    def body(x_ref, rw_ref, idx_ref, ew_ref, out_ref,
             acc_f32, rs_out, rs_in, ag_ref,
             rs_send_sems, rs_recv_sems, ag_send_sems, ag_recv_sems):
        me = lax.axis_index("i")
        peers = [lax.rem(me + off, N_DEV) for off in range(1, N_DEV)]
        send_order = (1, 0, 2)
        wait_order = (0, 2, 1)

        barrier = pltpu.get_barrier_semaphore()
        for p in peers:
            pl.semaphore_signal(barrier, inc=1, device_id=(p,),
                                device_id_type=pl.DeviceIdType.MESH)

        xf = x_ref[:, :]
        scores = jnp.dot(xf, rw_ref[:, :], preferred_element_type=jnp.float32)
        smax = jnp.max(scores, axis=1, keepdims=True)
        p_ = jnp.exp(scores - smax)
        probs = p_ / jnp.sum(p_, axis=1, keepdims=True)

        iota = lax.broadcasted_iota(jnp.int32, (n_tok, n_exp), 1)
        idx0 = idx_ref[:, 0:1]
        idx1 = idx_ref[:, 1:2]
        g0 = jnp.sum(jnp.where(iota == idx0, probs, 0.0), axis=1, keepdims=True)
        g1 = jnp.sum(jnp.where(iota == idx1, probs, 0.0), axis=1, keepdims=True)
        gs = g0 + g1
        w0 = g0 / gs
        w1 = g1 / gs

        gated = []
        for le in range(E_LOCAL):
            gid = me * E_LOCAL + le
            gate = (jnp.where(idx0 == gid, w0, 0.0)
                    + jnp.where(idx1 == gid, w1, 0.0))
            gated.append((xf * gate).astype(jnp.bfloat16))
        xg = jnp.concatenate(gated, axis=1)
        wcat = ew_ref[:, :, :].reshape(E_LOCAL * d, h).astype(jnp.bfloat16)
        acc = jnp.dot(xg, wcat, preferred_element_type=jnp.float32)
        acc4 = acc.reshape(N_DEV, q, h)
        acc_f32[:, :, :] = acc4
        rs_out[:, :, :] = acc4.astype(jnp.bfloat16)

        pl.semaphore_wait(barrier, N_DEV - 1)

        sends = []
        for k in send_order:
            p = peers[k]
            rdma = pltpu.make_async_remote_copy(
                src_ref=rs_out.at[p],
                dst_ref=rs_in.at[me],
                send_sem=rs_send_sems.at[k],
                recv_sem=rs_recv_sems.at[me],
                device_id=(p,),
                device_id_type=pl.DeviceIdType.MESH,
            )
            rdma.start()
            sends.append(rdma)

        red = acc_f32[me, :, :]
        for k in wait_order:
            p = peers[k]
            recv = pltpu.make_async_remote_copy(
                src_ref=rs_out.at[p],
                dst_ref=rs_in.at[p],
                send_sem=rs_send_sems.at[0],
                recv_sem=rs_recv_sems.at[p],
                device_id=(p,),
                device_id_type=pl.DeviceIdType.MESH,
            )
            recv.wait_recv()
            red = red + rs_in[p, :, :].astype(jnp.float32)

        ag_ref[me, :, :] = red.astype(jnp.bfloat16)
        out_ref[pl.ds(me * q, q), :] = red

        for k in send_order:
            p = peers[k]
            rdma = pltpu.make_async_remote_copy(
                src_ref=ag_ref.at[me],
                dst_ref=ag_ref.at[me],
                send_sem=ag_send_sems.at[k],
                recv_sem=ag_recv_sems.at[me],
                device_id=(p,),
                device_id_type=pl.DeviceIdType.MESH,
            )
            rdma.start()
            sends.append(rdma)

        for k in wait_order:
            p = peers[k]
            recv = pltpu.make_async_remote_copy(
                src_ref=ag_ref.at[me],
                dst_ref=ag_ref.at[p],
                send_sem=ag_send_sems.at[0],
                recv_sem=ag_recv_sems.at[p],
                device_id=(p,),
                device_id_type=pl.DeviceIdType.MESH,
            )
            recv.wait_recv()
            out_ref[pl.ds(p * q, q), :] = ag_ref[p, :, :].astype(jnp.float32)

        for rdma in sends:
            rdma.wait_send()

    return pl.pallas_call(
        body,
        out_shape=jax.ShapeDtypeStruct((n_tok, h), jnp.float32),
        in_specs=[pl.BlockSpec(memory_space=pltpu.VMEM)] * 4,
        out_specs=pl.BlockSpec(memory_space=pltpu.VMEM),
        scratch_shapes=[
            pltpu.VMEM((N_DEV, q, h), jnp.float32),
            pltpu.VMEM((N_DEV, q, h), jnp.bfloat16),
            pltpu.VMEM((N_DEV, q, h), jnp.bfloat16),
            pltpu.VMEM((N_DEV, q, h), jnp.bfloat16),
            pltpu.SemaphoreType.DMA((N_DEV - 1,)),
            pltpu.SemaphoreType.DMA((N_DEV,)),
            pltpu.SemaphoreType.DMA((N_DEV - 1,)),
            pltpu.SemaphoreType.DMA((N_DEV,)),
        ],
        compiler_params=pltpu.CompilerParams(collective_id=0),
    )(x, router_W, route_idx, expert_W)


# device time: 10430 ns/iter; 1.0826x vs baseline; 1.0826x over previous
import jax
import jax.numpy as jnp
from jax import lax
from jax.experimental import pallas as pl
from jax.experimental.pallas import tpu as pltpu

N_DEV = 4
E_LOCAL = 2


def kernel(x, router_W, route_idx, expert_W):
    n_tok, d = x.shape
    h = expert_W.shape[-1]
    n_exp = router_W.shape[-1]

    def body(x_ref, rw_ref, idx_ref, ew_ref, out_ref, comm_ref,
             send_sems, recv_sems):
        me = lax.axis_index("i")
        peers = [lax.rem(me + off, N_DEV) for off in range(1, N_DEV)]

        barrier = pltpu.get_barrier_semaphore()
        for p in peers:
            pl.semaphore_signal(barrier, inc=1, device_id=(p,),
                                device_id_type=pl.DeviceIdType.MESH)

        xf = x_ref[:, :]
        scores = jnp.dot(xf, rw_ref[:, :], preferred_element_type=jnp.float32)
        smax = jnp.max(scores, axis=1, keepdims=True)
        p_ = jnp.exp(scores - smax)
        probs = p_ / jnp.sum(p_, axis=1, keepdims=True)

        iota = lax.broadcasted_iota(jnp.int32, (n_tok, n_exp), 1)
        idx0 = idx_ref[:, 0:1]
        idx1 = idx_ref[:, 1:2]
        g0 = jnp.sum(jnp.where(iota == idx0, probs, 0.0), axis=1, keepdims=True)
        g1 = jnp.sum(jnp.where(iota == idx1, probs, 0.0), axis=1, keepdims=True)
        gs = g0 + g1
        w0 = g0 / gs
        w1 = g1 / gs

        gated = []
        for le in range(E_LOCAL):
            gid = me * E_LOCAL + le
            gate = (jnp.where(idx0 == gid, w0, 0.0)
                    + jnp.where(idx1 == gid, w1, 0.0))
            gated.append((xf * gate).astype(jnp.bfloat16))
        xg = jnp.concatenate(gated, axis=1)
        wcat = ew_ref[:, :, :].reshape(E_LOCAL * d, h).astype(jnp.bfloat16)
        acc = jnp.dot(xg, wcat, preferred_element_type=jnp.float32)

        comm_ref[me, :, :] = acc.astype(jnp.bfloat16)

        pl.semaphore_wait(barrier, N_DEV - 1)

        send_order = [1, 0, 2]
        sends = []
        for k in send_order:
            p = peers[k]
            rdma = pltpu.make_async_remote_copy(
                src_ref=comm_ref.at[me],
                dst_ref=comm_ref.at[me],
                send_sem=send_sems.at[k],
                recv_sem=recv_sems.at[me],
                device_id=(p,),
                device_id_type=pl.DeviceIdType.MESH,
            )
            rdma.start()
            sends.append(rdma)

        for k in (0, 2, 1):
            p = peers[k]
            recv = pltpu.make_async_remote_copy(
                src_ref=comm_ref.at[me],
                dst_ref=comm_ref.at[p],
                send_sem=send_sems.at[0],
                recv_sem=recv_sems.at[p],
                device_id=(p,),
                device_id_type=pl.DeviceIdType.MESH,
            )
            recv.wait_recv()
            acc = acc + comm_ref[p, :, :].astype(jnp.float32)

        out_ref[:, :] = acc

        for rdma in sends:
            rdma.wait_send()

    return pl.pallas_call(
        body,
        out_shape=jax.ShapeDtypeStruct((n_tok, h), jnp.float32),
        in_specs=[pl.BlockSpec(memory_space=pltpu.VMEM)] * 4,
        out_specs=pl.BlockSpec(memory_space=pltpu.VMEM),
        scratch_shapes=[
            pltpu.VMEM((N_DEV, n_tok, h), jnp.bfloat16),
            pltpu.SemaphoreType.DMA((N_DEV - 1,)),
            pltpu.SemaphoreType.DMA((N_DEV,)),
        ],
        compiler_params=pltpu.CompilerParams(collective_id=0),
    )(x, router_W, route_idx, expert_W)
